# Initial kernel scaffold; baseline (speedup 1.0000x reference)
import jax
import jax.numpy as jnp
from jax.experimental import pallas as pl


def kernel(features, params):
    raise NotImplementedError("write your pallas kernel here")


# speedup vs baseline: 1.0313x; 1.0313x over previous
import jax
import jax.numpy as jnp
import numpy as np
from jax.experimental import pallas as pl
from jax.experimental.pallas import tpu as pltpu

B = 1; CIN = 512; HID = 128; NC = 10; H = 180; W = 180; P = 200
HEADS = 8; FFN = 256; EPS = 1e-5
HW = H * W
HD = HID // HEADS
HEAD_DEFS = [("center", 2), ("height", 1), ("dim", 3), ("rot", 2), ("vel", 2), ("heatmap", NC)]

BH1 = 12
G1 = H // BH1
BH2 = 12
G2 = H // BH2
NKV = 2
KVH = HW // NKV
CK = 1080
NCHUNK = KVH // CK
SCALE = float(1.0 / np.sqrt(HD))
LOG2E = float(np.log2(np.e))


def _bf(x):
    return x.astype(jnp.bfloat16)


def _conv1_body(x_hbm, w_ref, b_ref, o_ref, xbuf, sem):
    i = pl.program_id(0)
    cp = pltpu.make_async_copy(x_hbm.at[pl.ds(i * BH1, BH1 + 2)], xbuf, sem)
    cp.start()
    cp.wait()
    acc = jnp.zeros((BH1 * W, HID), jnp.float32)
    for dy in range(3):
        for dx in range(3):
            lhs = xbuf[dy:dy + BH1, dx:dx + W, :].reshape(BH1 * W, CIN)
            acc += jnp.dot(lhs, w_ref[dy, dx],
                           preferred_element_type=jnp.float32)
    o_ref[...] = acc + b_ref[...]


def _conv1(xp, w, b):
    return pl.pallas_call(
        _conv1_body,
        grid=(G1,),
        in_specs=[
            pl.BlockSpec(memory_space=pl.ANY),
            pl.BlockSpec((3, 3, CIN, HID), lambda i: (0, 0, 0, 0)),
            pl.BlockSpec((1, HID), lambda i: (0, 0)),
        ],
        out_specs=pl.BlockSpec((BH1 * W, HID), lambda i: (i, 0)),
        out_shape=jax.ShapeDtypeStruct((HW, HID), jnp.float32),
        scratch_shapes=[pltpu.VMEM((BH1 + 2, W + 2, CIN), jnp.bfloat16),
                        pltpu.SemaphoreType.DMA],
        compiler_params=pltpu.CompilerParams(
            dimension_semantics=("arbitrary",)),
        name="conv1",
    )(xp, w, b)


def _convB_body(l_hbm, w2_ref, g2_ref, b2_ref, w3_ref, b3_ref, o_ref,
                lbuf, hbuf, sem):
    i = pl.program_id(0)
    cp = pltpu.make_async_copy(l_hbm.at[pl.ds(i * BH2, BH2 + 4)], lbuf, sem)
    cp.start()
    cp.wait()
    nh = BH2 + 2
    acc = jnp.zeros((nh * W, HID), jnp.float32)
    for dy in range(3):
        for dx in range(3):
            lhs = lbuf[dy:dy + nh, dx:dx + W, :].reshape(nh * W, HID)
            acc += jnp.dot(lhs, w2_ref[dy, dx],
                           preferred_element_type=jnp.float32)
    hb = jnp.maximum(acc * g2_ref[...] + b2_ref[...], 0.0)
    hbuf[:, 1:W + 1, :] = _bf(hb).reshape(nh, W, HID)
    hbuf[:, 0:1, :] = jnp.zeros((nh, 1, HID), jnp.bfloat16)
    hbuf[:, W + 1:W + 2, :] = jnp.zeros((nh, 1, HID), jnp.bfloat16)

    @pl.when(i == 0)
    def _():
        hbuf[0:1, :, :] = jnp.zeros((1, W + 2, HID), jnp.bfloat16)

    @pl.when(i == G2 - 1)
    def _():
        hbuf[nh - 1:nh, :, :] = jnp.zeros((1, W + 2, HID), jnp.bfloat16)
    accT = jnp.zeros((NC, BH2 * W), jnp.float32)
    for dy in range(3):
        for dx in range(3):
            rhs = hbuf[dy:dy + BH2, dx:dx + W, :].reshape(BH2 * W, HID)
            accT += jax.lax.dot_general(
                w3_ref[dy, dx], rhs, (((0,), (1,)), ((), ())),
                preferred_element_type=jnp.float32)
    o_ref[0] = accT + b3_ref[...]


def _convB(lp, w2, g2, b2, w3, b3):
    return pl.pallas_call(
        _convB_body,
        grid=(G2,),
        in_specs=[
            pl.BlockSpec(memory_space=pl.ANY),
            pl.BlockSpec((3, 3, HID, HID), lambda i: (0, 0, 0, 0)),
            pl.BlockSpec((1, HID), lambda i: (0, 0)),
            pl.BlockSpec((1, HID), lambda i: (0, 0)),
            pl.BlockSpec((3, 3, HID, NC), lambda i: (0, 0, 0, 0)),
            pl.BlockSpec((NC, 1), lambda i: (0, 0)),
        ],
        out_specs=pl.BlockSpec((1, NC, BH2 * W), lambda i: (i, 0, 0)),
        out_shape=jax.ShapeDtypeStruct((G2, NC, BH2 * W), jnp.float32),
        scratch_shapes=[pltpu.VMEM((BH2 + 4, W + 2, HID), jnp.bfloat16),
                        pltpu.VMEM((BH2 + 2, W + 2, HID), jnp.bfloat16),
                        pltpu.SemaphoreType.DMA],
        compiler_params=pltpu.CompilerParams(
            dimension_semantics=("arbitrary",)),
        name="conv23",
    )(lp, w2, g2, b2, w3, b3)


def _shift(x, s):
    z = jnp.zeros((NC, abs(s)), jnp.float32)
    if s > 0:
        return jnp.concatenate([z, x[:, :-s]], axis=1)
    return jnp.concatenate([x[:, -s:], z], axis=1)


def _nms_body(dh_ref, wm_ref, o_ref):
    h = jax.nn.sigmoid(dh_ref[...])
    cm = jnp.maximum(jnp.maximum(h, _shift(h, 1)), _shift(h, -1))
    wmax = jnp.maximum(jnp.maximum(cm, _shift(cm, W)), _shift(cm, -W))
    cls = jax.lax.broadcasted_iota(jnp.int32, (NC, HW), 0)
    keep = (cls >= 8) | ((h == wmax) & (wm_ref[...] > 0.0))
    o_ref[...] = jnp.where(keep, h, 0.0)


def _nms(dh, wmask):
    return pl.pallas_call(
        _nms_body,
        grid=(1,),
        in_specs=[pl.BlockSpec((NC, HW), lambda i: (0, 0)),
                  pl.BlockSpec((1, HW), lambda i: (0, 0))],
        out_specs=pl.BlockSpec((NC, HW), lambda i: (0, 0)),
        out_shape=jax.ShapeDtypeStruct((NC, HW), jnp.float32),
        name="nms",
    )(dh, wmask)


def _ln(x, g, b):
    m = jnp.mean(x, axis=-1, keepdims=True)
    d = x - m
    v = jnp.mean(d * d, axis=-1, keepdims=True)
    return d * jax.lax.rsqrt(v + EPS) * g + b


def _posembed(pos, w1t, b1, g, bb, w2t, b2):
    h = jnp.maximum((jnp.dot(pos, w1t, preferred_element_type=jnp.float32)
                     + b1) * g + bb, 0.0)
    return jnp.dot(_bf(h), w2t, preferred_element_type=jnp.float32) + b2


def _d0_body(idx_smem, kv_ref, cls_ref, qpos_ref, pr, o_q, o_qq, qf_s):
    qf_s[...] = jnp.zeros_like(qf_s)

    def gather(p, _):
        qf_s[pl.ds(p, 1), :] = kv_ref[pl.ds(idx_smem[p], 1), :]
        return 0
    jax.lax.fori_loop(0, P, gather, 0)

    onehot = (cls_ref[...]
              == jax.lax.broadcasted_iota(jnp.int32, (P, NC), 1))
    cls_emb = jnp.dot(jnp.where(onehot, 1.0, 0.0), pr["cls_wt"][...],
                      preferred_element_type=jnp.float32)
    q0 = qf_s[...] + cls_emb + pr["cls_b"][...]
    qpe = _posembed(qpos_ref[...], pr["spe_w1t"][...], pr["spe_b1"][...],
                    pr["spe_g"][...], pr["spe_b"][...],
                    pr["spe_w2t"][...], pr["spe_b2"][...])
    t = _bf(q0 + qpe)
    qh = jnp.dot(t, pr["sa_wqt"][...], preferred_element_type=jnp.float32) + pr["sa_bq"][...]
    kh = jnp.dot(t, pr["sa_wkt"][...], preferred_element_type=jnp.float32) + pr["sa_bk"][...]
    vh = jnp.dot(t, pr["sa_wvt"][...], preferred_element_type=jnp.float32) + pr["sa_bv"][...]
    qhb, khb, vhb = _bf(qh), _bf(kh), _bf(vh)
    outs = []
    for hh in range(HEADS):
        sl = slice(hh * HD, (hh + 1) * HD)
        s = jax.lax.dot_general(qhb[:, sl], khb[:, sl],
                                (((1,), (1,)), ((), ())),
                                preferred_element_type=jnp.float32) * SCALE
        m = jnp.max(s, axis=-1, keepdims=True)
        e = jnp.exp(s - m)
        a = e / jnp.sum(e, axis=-1, keepdims=True)
        outs.append(jnp.dot(_bf(a), vhb[:, sl],
                            preferred_element_type=jnp.float32))
    sa_o = jnp.concatenate(outs, axis=1)
    sa_o = jnp.dot(_bf(sa_o), pr["sa_wot"][...],
                   preferred_element_type=jnp.float32) + pr["sa_bo"][...]
    q1 = _ln(q0 + sa_o, pr["n1_g"][...], pr["n1_b"][...])
    qq = jnp.dot(_bf(q1 + qpe), pr["ca_wqt"][...],
                 preferred_element_type=jnp.float32) + pr["ca_bq"][...]
    o_q[...] = q1
    o_qq[...] = qq


def _d1_body(kv_ref, qq_ref, bev_ref, pr, o_acc, o_ml, ml_s):
    qqb = _bf(qq_ref[...] * (SCALE * LOG2E))
    ml_s[:, 0:HEADS] = jnp.full((P, HEADS), -1e30, jnp.float32)
    ml_s[:, HEADS:2 * HEADS] = jnp.zeros((P, HEADS), jnp.float32)
    o_acc[0] = jnp.zeros((P, HID), jnp.float32)

    w1t = pr["cpe_w1t"][...]
    b1 = pr["cpe_b1"][...]
    g = pr["cpe_g"][...]
    bb = pr["cpe_b"][...]
    w2t = pr["cpe_w2t"][...]
    b2 = pr["cpe_b2"][...]
    wkt = pr["ca_wkt"][...]
    bk = pr["ca_bk"][...]
    wvt = pr["ca_wvt"][...]
    bv = pr["ca_bv"][...]

    for c in range(NCHUNK):
        kvc = kv_ref[pl.ds(c * CK, CK), :]
        kpe = _posembed(bev_ref[pl.ds(c * CK, CK), :], w1t, b1, g, bb,
                        w2t, b2)
        kvp = _bf(kvc + kpe)
        kk = _bf(jnp.dot(kvp, wkt, preferred_element_type=jnp.float32) + bk)
        vv = _bf(jnp.dot(kvp, wvt, preferred_element_type=jnp.float32) + bv)
        for hh in range(HEADS):
            sl = slice(hh * HD, (hh + 1) * HD)
            s = jax.lax.dot_general(qqb[:, sl], kk[:, sl],
                                    (((1,), (1,)), ((), ())),
                                    preferred_element_type=jnp.float32)
            mo = ml_s[:, hh:hh + 1]
            lo = ml_s[:, HEADS + hh:HEADS + hh + 1]
            ao = o_acc[0, :, sl]
            mn = jnp.maximum(mo, jnp.max(s, axis=-1, keepdims=True))
            p = jnp.exp2(s - mn)
            al = jnp.exp2(mo - mn)
            ml_s[:, hh:hh + 1] = mn
            ml_s[:, HEADS + hh:HEADS + hh + 1] = (
                lo * al + jnp.sum(p, axis=-1, keepdims=True))
            o_acc[0, :, sl] = ao * al + jnp.dot(
                _bf(p), vv[:, sl], preferred_element_type=jnp.float32)
    o_ml[0] = ml_s[...]


def _d2_body(q_ref, qpos_ref, acc_ref, ml_ref, pr, o_ref):
    m0 = ml_ref[0, :, 0:HEADS]
    m1 = ml_ref[1, :, 0:HEADS]
    l0 = ml_ref[0, :, HEADS:2 * HEADS]
    l1 = ml_ref[1, :, HEADS:2 * HEADS]
    mg = jnp.maximum(m0, m1)
    w0 = jnp.exp2(m0 - mg)
    w1 = jnp.exp2(m1 - mg)
    lg = l0 * w0 + l1 * w1
    cols = []
    for hh in range(HEADS):
        sl = slice(hh * HD, (hh + 1) * HD)
        num = (acc_ref[0, :, sl] * w0[:, hh:hh + 1]
               + acc_ref[1, :, sl] * w1[:, hh:hh + 1])
        cols.append(num / lg[:, hh:hh + 1])
    ca_o = jnp.concatenate(cols, axis=1)
    ca_o = jnp.dot(_bf(ca_o), pr["ca_wot"][...],
                   preferred_element_type=jnp.float32) + pr["ca_bo"][...]
    q2 = _ln(q_ref[...] + ca_o, pr["n2_g"][...], pr["n2_b"][...])
    f = jnp.maximum(jnp.dot(_bf(q2), pr["ffn_w1t"][...],
                            preferred_element_type=jnp.float32)
                    + pr["ffn_b1"][...], 0.0)
    f = jnp.dot(_bf(f), pr["ffn_w2t"][...],
                preferred_element_type=jnp.float32) + pr["ffn_b2"][...]
    q3 = _ln(q2 + f, pr["n3_g"][...], pr["n3_b"][...])
    q3b = _bf(q3)
    pieces = []
    for name, oc in HEAD_DEFS:
        hh = jnp.maximum(
            jnp.dot(q3b, pr[name + "_w1t"][...],
                    preferred_element_type=jnp.float32)
            * pr[name + "_g"][...] + pr[name + "_b"][...], 0.0)
        o = jnp.dot(_bf(hh), pr[name + "_w2t"][...],
                    preferred_element_type=jnp.float32) + pr[name + "_b2"][...]
        if name == "center":
            o = o + qpos_ref[...]
        pieces.append(o)
    o_ref[...] = jnp.concatenate(pieces, axis=1)


def _full_specs(tree):
    return jax.tree.map(
        lambda a: pl.BlockSpec(a.shape, lambda *_: (0,) * a.ndim), tree)


def _decoder(kv, idx_pad, cls2d, qpos, bev, prep):
    q1, qq = pl.pallas_call(
        _d0_body,
        grid=(1,),
        in_specs=[
            pl.BlockSpec(memory_space=pltpu.SMEM),
            pl.BlockSpec((HW, HID), lambda i: (0, 0)),
            pl.BlockSpec((P, 1), lambda i: (0, 0)),
            pl.BlockSpec((P, 2), lambda i: (0, 0)),
            _full_specs(prep),
        ],
        out_specs=[pl.BlockSpec((P, HID), lambda i: (0, 0))] * 2,
        out_shape=[jax.ShapeDtypeStruct((P, HID), jnp.float32)] * 2,
        scratch_shapes=[pltpu.VMEM((P, HID), jnp.float32)],
        name="dec_gather_selfattn",
    )(idx_pad, kv, cls2d, qpos, prep)

    acc, ml = pl.pallas_call(
        _d1_body,
        grid=(NKV,),
        in_specs=[
            pl.BlockSpec((KVH, HID), lambda i: (i, 0)),
            pl.BlockSpec((P, HID), lambda i: (0, 0)),
            pl.BlockSpec((KVH, 2), lambda i: (i, 0)),
            _full_specs(prep),
        ],
        out_specs=[pl.BlockSpec((1, P, HID), lambda i: (i, 0, 0)),
                   pl.BlockSpec((1, P, 2 * HEADS), lambda i: (i, 0, 0))],
        out_shape=[jax.ShapeDtypeStruct((NKV, P, HID), jnp.float32),
                   jax.ShapeDtypeStruct((NKV, P, 2 * HEADS), jnp.float32)],
        scratch_shapes=[pltpu.VMEM((P, 2 * HEADS), jnp.float32)],
        compiler_params=pltpu.CompilerParams(
            dimension_semantics=("arbitrary",)),
        name="dec_crossattn",
    )(kv, qq, bev, prep)

    preds = pl.pallas_call(
        _d2_body,
        grid=(1,),
        in_specs=[
            pl.BlockSpec((P, HID), lambda i: (0, 0)),
            pl.BlockSpec((P, 2), lambda i: (0, 0)),
            pl.BlockSpec((NKV, P, HID), lambda i: (0, 0, 0)),
            pl.BlockSpec((NKV, P, 2 * HEADS), lambda i: (0, 0, 0)),
            _full_specs(prep),
        ],
        out_specs=pl.BlockSpec((P, 20), lambda i: (0, 0)),
        out_shape=jax.ShapeDtypeStruct((P, 20), jnp.float32),
        name="dec_ffn_heads",
    )(q1, qpos, acc, ml, prep)
    return preds


def kernel(features, params):
    x = features[0].transpose(1, 2, 0)
    xp = jnp.pad(x, ((1, 1), (1, 1), (0, 0))).astype(jnp.bfloat16)
    w1 = _bf(params["shared_w"].transpose(2, 3, 1, 0))
    b1 = params["shared_b"].reshape(1, HID)
    lidar = _conv1(xp, w1, b1)

    lp = jnp.pad(_bf(lidar).reshape(H, W, HID), ((2, 2), (1, 1), (0, 0)))
    w2 = _bf(params["blk_w"].transpose(2, 3, 1, 0))
    w3 = _bf(params["hm_w"].transpose(2, 3, 1, 0))
    dh = _convB(lp, w2, params["blk_g"].reshape(1, HID),
                params["blk_b"].reshape(1, HID), w3,
                params["hm_b"].reshape(NC, 1))
    dh = dh.transpose(1, 0, 2).reshape(NC, HW)

    ww = jnp.arange(HW, dtype=jnp.int32) % W
    hh = jnp.arange(HW, dtype=jnp.int32) // W
    wmask = (((ww >= 1) & (ww <= W - 2) & (hh >= 1) & (hh <= H - 2))
             .astype(jnp.float32).reshape(1, HW))
    masked = _nms(dh, wmask)

    _, top = jax.lax.top_k(masked.reshape(1, NC * HW), P)
    top = top[0]
    top_cls = (top // HW).astype(jnp.int32)
    top_idx = (top % HW).astype(jnp.int32)
    qpos = jnp.stack([(top_idx // W).astype(jnp.float32) + 0.5,
                      (top_idx % W).astype(jnp.float32) + 0.5], axis=1)
    bev = jnp.stack([(hh).astype(jnp.float32) + 0.5,
                     (ww).astype(jnp.float32) + 0.5], axis=1)

    p = params
    prep = {
        "cls_wt": jnp.transpose(p["cls_w"]),
        "cls_b": p["cls_b"].reshape(1, HID),
        "spe_w1t": p["self_pe"]["w1"].T.reshape(2, HID),
        "spe_b1": p["self_pe"]["b1"].reshape(1, HID),
        "spe_g": p["self_pe"]["g"].reshape(1, HID),
        "spe_b": p["self_pe"]["b"].reshape(1, HID),
        "spe_w2t": _bf(p["self_pe"]["w2"].T),
        "spe_b2": p["self_pe"]["b2"].reshape(1, HID),
        "cpe_w1t": p["cross_pe"]["w1"].T.reshape(2, HID),
        "cpe_b1": p["cross_pe"]["b1"].reshape(1, HID),
        "cpe_g": p["cross_pe"]["g"].reshape(1, HID),
        "cpe_b": p["cross_pe"]["b"].reshape(1, HID),
        "cpe_w2t": _bf(p["cross_pe"]["w2"].T),
        "cpe_b2": p["cross_pe"]["b2"].reshape(1, HID),
        "n1_g": p["n1_g"].reshape(1, HID), "n1_b": p["n1_b"].reshape(1, HID),
        "n2_g": p["n2_g"].reshape(1, HID), "n2_b": p["n2_b"].reshape(1, HID),
        "n3_g": p["n3_g"].reshape(1, HID), "n3_b": p["n3_b"].reshape(1, HID),
        "ffn_w1t": _bf(p["ffn_w1"].T), "ffn_b1": p["ffn_b1"].reshape(1, FFN),
        "ffn_w2t": _bf(p["ffn_w2"].T), "ffn_b2": p["ffn_b2"].reshape(1, HID),
    }
    for nm in ("sa", "ca"):
        a = p[nm]
        prep[nm + "_wqt"] = _bf(a["wq"].T)
        prep[nm + "_bq"] = a["bq"].reshape(1, HID)
        prep[nm + "_wkt"] = _bf(a["wk"].T)
        prep[nm + "_bk"] = a["bk"].reshape(1, HID)
        prep[nm + "_wvt"] = _bf(a["wv"].T)
        prep[nm + "_bv"] = a["bv"].reshape(1, HID)
        prep[nm + "_wot"] = _bf(a["wo"].T)
        prep[nm + "_bo"] = a["bo"].reshape(1, HID)
    for name, oc in HEAD_DEFS:
        hp = p["heads"][name]
        prep[name + "_w1t"] = _bf(hp["w1"].T)
        prep[name + "_g"] = hp["g"].reshape(1, 64)
        prep[name + "_b"] = hp["b"].reshape(1, 64)
        prep[name + "_w2t"] = _bf(hp["w2"].T)
        prep[name + "_b2"] = hp["b2"].reshape(1, oc)

    preds = _decoder(lidar, top_idx, top_cls.reshape(P, 1), qpos, bev, prep)

    qhs = jnp.take(masked, top_idx, axis=1)
    return (preds.T[None], qhs[None], dh.reshape(1, NC, H, W))


# speedup vs baseline: 1.0813x; 1.0485x over previous
import jax
import jax.numpy as jnp
import numpy as np
from jax.experimental import pallas as pl
from jax.experimental.pallas import tpu as pltpu

B = 1; CIN = 512; HID = 128; NC = 10; H = 180; W = 180; P = 200
HEADS = 8; FFN = 256; EPS = 1e-5
HW = H * W
HD = HID // HEADS
HEAD_DEFS = [("center", 2), ("height", 1), ("dim", 3), ("rot", 2), ("vel", 2), ("heatmap", NC)]

BH1 = 12
G1 = H // BH1
BH2 = 12
G2 = H // BH2
NKV = 2
KVH = HW // NKV
CK = 1080
NCHUNK = KVH // CK
SCALE = float(1.0 / np.sqrt(HD))
LOG2E = float(np.log2(np.e))


def _bf(x):
    return x.astype(jnp.bfloat16)


def _conv1_body(x_hbm, w_ref, b_ref, o_ref, xbuf, sem):
    i = pl.program_id(0)
    cp = pltpu.make_async_copy(x_hbm.at[pl.ds(i * BH1, BH1 + 2)], xbuf, sem)
    cp.start()
    cp.wait()
    acc = jnp.zeros((BH1 * W, HID), jnp.float32)
    for dy in range(3):
        for dx in range(3):
            lhs = xbuf[dy:dy + BH1, dx:dx + W, :].reshape(BH1 * W, CIN)
            acc += jnp.dot(lhs, w_ref[dy, dx],
                           preferred_element_type=jnp.float32)
    o_ref[...] = acc + b_ref[...]


def _conv1(xp, w, b):
    return pl.pallas_call(
        _conv1_body,
        grid=(G1,),
        in_specs=[
            pl.BlockSpec(memory_space=pl.ANY),
            pl.BlockSpec((3, 3, CIN, HID), lambda i: (0, 0, 0, 0)),
            pl.BlockSpec((1, HID), lambda i: (0, 0)),
        ],
        out_specs=pl.BlockSpec((BH1 * W, HID), lambda i: (i, 0)),
        out_shape=jax.ShapeDtypeStruct((HW, HID), jnp.float32),
        scratch_shapes=[pltpu.VMEM((BH1 + 2, W + 2, CIN), jnp.bfloat16),
                        pltpu.SemaphoreType.DMA],
        compiler_params=pltpu.CompilerParams(
            dimension_semantics=("arbitrary",)),
        name="conv1",
    )(xp, w, b)


def _convB_body(l_hbm, w2_ref, g2_ref, b2_ref, w3_ref, b3_ref, o_ref,
                lbuf, hbuf, sem):
    i = pl.program_id(0)
    cp = pltpu.make_async_copy(l_hbm.at[pl.ds(i * BH2, BH2 + 4)], lbuf, sem)
    cp.start()
    cp.wait()
    nh = BH2 + 2
    acc = jnp.zeros((nh * W, HID), jnp.float32)
    for dy in range(3):
        for dx in range(3):
            lhs = lbuf[dy:dy + nh, dx:dx + W, :].reshape(nh * W, HID)
            acc += jnp.dot(lhs, w2_ref[dy, dx],
                           preferred_element_type=jnp.float32)
    hb = jnp.maximum(acc * g2_ref[...] + b2_ref[...], 0.0)
    hbuf[:, 1:W + 1, :] = _bf(hb).reshape(nh, W, HID)
    hbuf[:, 0:1, :] = jnp.zeros((nh, 1, HID), jnp.bfloat16)
    hbuf[:, W + 1:W + 2, :] = jnp.zeros((nh, 1, HID), jnp.bfloat16)

    @pl.when(i == 0)
    def _():
        hbuf[0:1, :, :] = jnp.zeros((1, W + 2, HID), jnp.bfloat16)

    @pl.when(i == G2 - 1)
    def _():
        hbuf[nh - 1:nh, :, :] = jnp.zeros((1, W + 2, HID), jnp.bfloat16)
    accT = jnp.zeros((NC, BH2 * W), jnp.float32)
    for dy in range(3):
        for dx in range(3):
            rhs = hbuf[dy:dy + BH2, dx:dx + W, :].reshape(BH2 * W, HID)
            accT += jax.lax.dot_general(
                w3_ref[dy, dx], rhs, (((0,), (1,)), ((), ())),
                preferred_element_type=jnp.float32)
    o_ref[0] = accT + b3_ref[...]


def _convB(lp, w2, g2, b2, w3, b3):
    return pl.pallas_call(
        _convB_body,
        grid=(G2,),
        in_specs=[
            pl.BlockSpec(memory_space=pl.ANY),
            pl.BlockSpec((3, 3, HID, HID), lambda i: (0, 0, 0, 0)),
            pl.BlockSpec((1, HID), lambda i: (0, 0)),
            pl.BlockSpec((1, HID), lambda i: (0, 0)),
            pl.BlockSpec((3, 3, HID, NC), lambda i: (0, 0, 0, 0)),
            pl.BlockSpec((NC, 1), lambda i: (0, 0)),
        ],
        out_specs=pl.BlockSpec((1, NC, BH2 * W), lambda i: (i, 0, 0)),
        out_shape=jax.ShapeDtypeStruct((G2, NC, BH2 * W), jnp.float32),
        scratch_shapes=[pltpu.VMEM((BH2 + 4, W + 2, HID), jnp.bfloat16),
                        pltpu.VMEM((BH2 + 2, W + 2, HID), jnp.bfloat16),
                        pltpu.SemaphoreType.DMA],
        compiler_params=pltpu.CompilerParams(
            dimension_semantics=("arbitrary",)),
        name="conv23",
    )(lp, w2, g2, b2, w3, b3)


def _shift(x, s):
    z = jnp.zeros((NC, abs(s)), jnp.float32)
    if s > 0:
        return jnp.concatenate([z, x[:, :-s]], axis=1)
    return jnp.concatenate([x[:, -s:], z], axis=1)


def _nms_body(dh_ref, wm_ref, o_ref):
    h = jax.nn.sigmoid(dh_ref[...])
    cm = jnp.maximum(jnp.maximum(h, _shift(h, 1)), _shift(h, -1))
    wmax = jnp.maximum(jnp.maximum(cm, _shift(cm, W)), _shift(cm, -W))
    cls = jax.lax.broadcasted_iota(jnp.int32, (NC, HW), 0)
    keep = (cls >= 8) | ((h == wmax) & (wm_ref[...] > 0.0))
    o_ref[...] = jnp.where(keep, h, 0.0)


def _nms(dh, wmask):
    return pl.pallas_call(
        _nms_body,
        grid=(1,),
        in_specs=[pl.BlockSpec((NC, HW), lambda i: (0, 0)),
                  pl.BlockSpec((1, HW), lambda i: (0, 0))],
        out_specs=pl.BlockSpec((NC, HW), lambda i: (0, 0)),
        out_shape=jax.ShapeDtypeStruct((NC, HW), jnp.float32),
        name="nms",
    )(dh, wmask)


TKR = 2536
TKCAP = 256


def _topk_body(a_ref, at_ref, o_vals, o_rid, rc_vmem, rc_smem, nq_s, sem):
    o_vals[...] = jnp.full((TKCAP, 128), -1.0, jnp.float32)
    o_rid[...] = jnp.zeros((TKCAP, 128), jnp.float32)

    def sbody(_, lh):
        lo, hi = lh
        mid = (lo + hi) // 2
        ai = pltpu.bitcast(a_ref[...], jnp.int32)
        cnt = jnp.sum(jnp.where(ai > mid, 1.0, 0.0))
        big = cnt >= float(P)
        return (jnp.where(big, mid, lo), jnp.where(big, hi, mid))

    lo, hi = jax.lax.fori_loop(
        0, 31, sbody, (jnp.int32(0), jnp.int32(0x3F800000)))
    tbits = hi

    atb = pltpu.bitcast(at_ref[...], jnp.int32)
    rc = jnp.sum(jnp.where(atb >= tbits, 1.0, 0.0), axis=0, keepdims=True)
    rc_vmem[...] = jnp.zeros((8, TKR + 24), jnp.float32)
    rc_vmem[0:1, 0:TKR] = rc
    cp = pltpu.make_async_copy(rc_vmem, rc_smem, sem)
    cp.start()
    cp.wait()

    nq_s[0] = 0

    def scan(r, _):
        @pl.when(rc_smem[0, r] > 0.5)
        def _():
            q = nq_s[0]

            @pl.when(q < TKCAP)
            def _():
                o_vals[pl.ds(q, 1), :] = a_ref[pl.ds(r, 1), :]
                o_rid[pl.ds(q, 1), :] = jnp.full(
                    (1, 128), (r * 128).astype(jnp.float32), jnp.float32)
            nq_s[0] = q + 1
        return 0

    jax.lax.fori_loop(0, TKR, scan, 0)


def _topk_stage(a2, a2t):
    return pl.pallas_call(
        _topk_body,
        grid=(1,),
        in_specs=[pl.BlockSpec((TKR, 128), lambda i: (0, 0)),
                  pl.BlockSpec((128, TKR), lambda i: (0, 0))],
        out_specs=[pl.BlockSpec((TKCAP, 128), lambda i: (0, 0)),
                   pl.BlockSpec((TKCAP, 128), lambda i: (0, 0))],
        out_shape=[jax.ShapeDtypeStruct((TKCAP, 128), jnp.float32),
                   jax.ShapeDtypeStruct((TKCAP, 128), jnp.float32)],
        scratch_shapes=[pltpu.VMEM((8, TKR + 24), jnp.float32),
                        pltpu.SMEM((8, TKR + 24), jnp.float32),
                        pltpu.SMEM((1,), jnp.int32),
                        pltpu.SemaphoreType.DMA],
        name="topk_stage",
    )(a2, a2t)


def _ln(x, g, b):
    m = jnp.mean(x, axis=-1, keepdims=True)
    d = x - m
    v = jnp.mean(d * d, axis=-1, keepdims=True)
    return d * jax.lax.rsqrt(v + EPS) * g + b


def _posembed(pos, w1t, b1, g, bb, w2t, b2):
    h = jnp.maximum((jnp.dot(pos, w1t, preferred_element_type=jnp.float32)
                     + b1) * g + bb, 0.0)
    return jnp.dot(_bf(h), w2t, preferred_element_type=jnp.float32) + b2


def _d0_body(idx_smem, kv_ref, cls_ref, qpos_ref, pr, o_q, o_qq, qf_s):
    qf_s[...] = jnp.zeros_like(qf_s)

    def gather(p, _):
        qf_s[pl.ds(p, 1), :] = kv_ref[pl.ds(idx_smem[p], 1), :]
        return 0
    jax.lax.fori_loop(0, P, gather, 0)

    onehot = (cls_ref[...]
              == jax.lax.broadcasted_iota(jnp.int32, (P, NC), 1))
    cls_emb = jnp.dot(jnp.where(onehot, 1.0, 0.0), pr["cls_wt"][...],
                      preferred_element_type=jnp.float32)
    q0 = qf_s[...] + cls_emb + pr["cls_b"][...]
    qpe = _posembed(qpos_ref[...], pr["spe_w1t"][...], pr["spe_b1"][...],
                    pr["spe_g"][...], pr["spe_b"][...],
                    pr["spe_w2t"][...], pr["spe_b2"][...])
    t = _bf(q0 + qpe)
    qh = jnp.dot(t, pr["sa_wqt"][...], preferred_element_type=jnp.float32) + pr["sa_bq"][...]
    kh = jnp.dot(t, pr["sa_wkt"][...], preferred_element_type=jnp.float32) + pr["sa_bk"][...]
    vh = jnp.dot(t, pr["sa_wvt"][...], preferred_element_type=jnp.float32) + pr["sa_bv"][...]
    qhb, khb, vhb = _bf(qh), _bf(kh), _bf(vh)
    outs = []
    for hh in range(HEADS):
        sl = slice(hh * HD, (hh + 1) * HD)
        s = jax.lax.dot_general(qhb[:, sl], khb[:, sl],
                                (((1,), (1,)), ((), ())),
                                preferred_element_type=jnp.float32) * SCALE
        m = jnp.max(s, axis=-1, keepdims=True)
        e = jnp.exp(s - m)
        a = e / jnp.sum(e, axis=-1, keepdims=True)
        outs.append(jnp.dot(_bf(a), vhb[:, sl],
                            preferred_element_type=jnp.float32))
    sa_o = jnp.concatenate(outs, axis=1)
    sa_o = jnp.dot(_bf(sa_o), pr["sa_wot"][...],
                   preferred_element_type=jnp.float32) + pr["sa_bo"][...]
    q1 = _ln(q0 + sa_o, pr["n1_g"][...], pr["n1_b"][...])
    qq = jnp.dot(_bf(q1 + qpe), pr["ca_wqt"][...],
                 preferred_element_type=jnp.float32) + pr["ca_bq"][...]
    o_q[...] = q1
    o_qq[...] = qq


def _d1_body(kv_ref, qq_ref, bev_ref, pr, o_acc, o_ml, ml_s):
    qqb = _bf(qq_ref[...] * (SCALE * LOG2E))
    ml_s[:, 0:HEADS] = jnp.full((P, HEADS), -1e30, jnp.float32)
    ml_s[:, HEADS:2 * HEADS] = jnp.zeros((P, HEADS), jnp.float32)
    o_acc[0] = jnp.zeros((P, HID), jnp.float32)

    w1t = pr["cpe_w1t"][...]
    b1 = pr["cpe_b1"][...]
    g = pr["cpe_g"][...]
    bb = pr["cpe_b"][...]
    w2t = pr["cpe_w2t"][...]
    b2 = pr["cpe_b2"][...]
    wkt = pr["ca_wkt"][...]
    bk = pr["ca_bk"][...]
    wvt = pr["ca_wvt"][...]
    bv = pr["ca_bv"][...]

    for c in range(NCHUNK):
        kvc = kv_ref[pl.ds(c * CK, CK), :]
        kpe = _posembed(bev_ref[pl.ds(c * CK, CK), :], w1t, b1, g, bb,
                        w2t, b2)
        kvp = _bf(kvc + kpe)
        kk = _bf(jnp.dot(kvp, wkt, preferred_element_type=jnp.float32) + bk)
        vv = _bf(jnp.dot(kvp, wvt, preferred_element_type=jnp.float32) + bv)
        for hh in range(HEADS):
            sl = slice(hh * HD, (hh + 1) * HD)
            s = jax.lax.dot_general(qqb[:, sl], kk[:, sl],
                                    (((1,), (1,)), ((), ())),
                                    preferred_element_type=jnp.float32)
            mo = ml_s[:, hh:hh + 1]
            lo = ml_s[:, HEADS + hh:HEADS + hh + 1]
            ao = o_acc[0, :, sl]
            mn = jnp.maximum(mo, jnp.max(s, axis=-1, keepdims=True))
            p = jnp.exp2(s - mn)
            al = jnp.exp2(mo - mn)
            ml_s[:, hh:hh + 1] = mn
            ml_s[:, HEADS + hh:HEADS + hh + 1] = (
                lo * al + jnp.sum(p, axis=-1, keepdims=True))
            o_acc[0, :, sl] = ao * al + jnp.dot(
                _bf(p), vv[:, sl], preferred_element_type=jnp.float32)
    o_ml[0] = ml_s[...]


def _d2_body(q_ref, qpos_ref, acc_ref, ml_ref, pr, o_ref):
    m0 = ml_ref[0, :, 0:HEADS]
    m1 = ml_ref[1, :, 0:HEADS]
    l0 = ml_ref[0, :, HEADS:2 * HEADS]
    l1 = ml_ref[1, :, HEADS:2 * HEADS]
    mg = jnp.maximum(m0, m1)
    w0 = jnp.exp2(m0 - mg)
    w1 = jnp.exp2(m1 - mg)
    lg = l0 * w0 + l1 * w1
    cols = []
    for hh in range(HEADS):
        sl = slice(hh * HD, (hh + 1) * HD)
        num = (acc_ref[0, :, sl] * w0[:, hh:hh + 1]
               + acc_ref[1, :, sl] * w1[:, hh:hh + 1])
        cols.append(num / lg[:, hh:hh + 1])
    ca_o = jnp.concatenate(cols, axis=1)
    ca_o = jnp.dot(_bf(ca_o), pr["ca_wot"][...],
                   preferred_element_type=jnp.float32) + pr["ca_bo"][...]
    q2 = _ln(q_ref[...] + ca_o, pr["n2_g"][...], pr["n2_b"][...])
    f = jnp.maximum(jnp.dot(_bf(q2), pr["ffn_w1t"][...],
                            preferred_element_type=jnp.float32)
                    + pr["ffn_b1"][...], 0.0)
    f = jnp.dot(_bf(f), pr["ffn_w2t"][...],
                preferred_element_type=jnp.float32) + pr["ffn_b2"][...]
    q3 = _ln(q2 + f, pr["n3_g"][...], pr["n3_b"][...])
    q3b = _bf(q3)
    pieces = []
    for name, oc in HEAD_DEFS:
        hh = jnp.maximum(
            jnp.dot(q3b, pr[name + "_w1t"][...],
                    preferred_element_type=jnp.float32)
            * pr[name + "_g"][...] + pr[name + "_b"][...], 0.0)
        o = jnp.dot(_bf(hh), pr[name + "_w2t"][...],
                    preferred_element_type=jnp.float32) + pr[name + "_b2"][...]
        if name == "center":
            o = o + qpos_ref[...]
        pieces.append(o)
    o_ref[...] = jnp.concatenate(pieces, axis=1)


def _full_specs(tree):
    return jax.tree.map(
        lambda a: pl.BlockSpec(a.shape, lambda *_: (0,) * a.ndim), tree)


def _decoder(kv, idx_pad, cls2d, qpos, bev, prep):
    q1, qq = pl.pallas_call(
        _d0_body,
        grid=(1,),
        in_specs=[
            pl.BlockSpec(memory_space=pltpu.SMEM),
            pl.BlockSpec((HW, HID), lambda i: (0, 0)),
            pl.BlockSpec((P, 1), lambda i: (0, 0)),
            pl.BlockSpec((P, 2), lambda i: (0, 0)),
            _full_specs(prep),
        ],
        out_specs=[pl.BlockSpec((P, HID), lambda i: (0, 0))] * 2,
        out_shape=[jax.ShapeDtypeStruct((P, HID), jnp.float32)] * 2,
        scratch_shapes=[pltpu.VMEM((P, HID), jnp.float32)],
        name="dec_gather_selfattn",
    )(idx_pad, kv, cls2d, qpos, prep)

    acc, ml = pl.pallas_call(
        _d1_body,
        grid=(NKV,),
        in_specs=[
            pl.BlockSpec((KVH, HID), lambda i: (i, 0)),
            pl.BlockSpec((P, HID), lambda i: (0, 0)),
            pl.BlockSpec((KVH, 2), lambda i: (i, 0)),
            _full_specs(prep),
        ],
        out_specs=[pl.BlockSpec((1, P, HID), lambda i: (i, 0, 0)),
                   pl.BlockSpec((1, P, 2 * HEADS), lambda i: (i, 0, 0))],
        out_shape=[jax.ShapeDtypeStruct((NKV, P, HID), jnp.float32),
                   jax.ShapeDtypeStruct((NKV, P, 2 * HEADS), jnp.float32)],
        scratch_shapes=[pltpu.VMEM((P, 2 * HEADS), jnp.float32)],
        compiler_params=pltpu.CompilerParams(
            dimension_semantics=("arbitrary",)),
        name="dec_crossattn",
    )(kv, qq, bev, prep)

    preds = pl.pallas_call(
        _d2_body,
        grid=(1,),
        in_specs=[
            pl.BlockSpec((P, HID), lambda i: (0, 0)),
            pl.BlockSpec((P, 2), lambda i: (0, 0)),
            pl.BlockSpec((NKV, P, HID), lambda i: (0, 0, 0)),
            pl.BlockSpec((NKV, P, 2 * HEADS), lambda i: (0, 0, 0)),
            _full_specs(prep),
        ],
        out_specs=pl.BlockSpec((P, 20), lambda i: (0, 0)),
        out_shape=jax.ShapeDtypeStruct((P, 20), jnp.float32),
        name="dec_ffn_heads",
    )(q1, qpos, acc, ml, prep)
    return preds


def kernel(features, params):
    x = features[0].astype(jnp.bfloat16).transpose(1, 2, 0)
    xp = jnp.pad(x, ((1, 1), (1, 1), (0, 0)))
    w1 = _bf(params["shared_w"].transpose(2, 3, 1, 0))
    b1 = params["shared_b"].reshape(1, HID)
    lidar = _conv1(xp, w1, b1)

    lp = jnp.pad(_bf(lidar).reshape(H, W, HID), ((2, 2), (1, 1), (0, 0)))
    w2 = _bf(params["blk_w"].transpose(2, 3, 1, 0))
    w3 = _bf(params["hm_w"].transpose(2, 3, 1, 0))
    dh = _convB(lp, w2, params["blk_g"].reshape(1, HID),
                params["blk_b"].reshape(1, HID), w3,
                params["hm_b"].reshape(NC, 1))
    dh = dh.transpose(1, 0, 2).reshape(NC, HW)

    ww = jnp.arange(HW, dtype=jnp.int32) % W
    hh = jnp.arange(HW, dtype=jnp.int32) // W
    wmask = (((ww >= 1) & (ww <= W - 2) & (hh >= 1) & (hh <= H - 2))
             .astype(jnp.float32).reshape(1, HW))
    masked = _nms(dh, wmask)

    flat = masked.reshape(NC * HW)
    flatp = jnp.concatenate(
        [flat, jnp.zeros((TKR * 128 - NC * HW,), jnp.float32)])
    a2 = flatp.reshape(TKR, 128)
    sv, srid = _topk_stage(a2, a2.T)
    _, sp = jax.lax.top_k(sv.reshape(1, TKCAP * 128), P)
    sp = sp[0]
    base = srid[:, 0].astype(jnp.int32)
    top = jnp.take(base, sp // 128) + (sp % 128)
    top_cls = (top // HW).astype(jnp.int32)
    top_idx = (top % HW).astype(jnp.int32)
    qpos = jnp.stack([(top_idx // W).astype(jnp.float32) + 0.5,
                      (top_idx % W).astype(jnp.float32) + 0.5], axis=1)
    bev = jnp.stack([(hh).astype(jnp.float32) + 0.5,
                     (ww).astype(jnp.float32) + 0.5], axis=1)

    p = params
    prep = {
        "cls_wt": jnp.transpose(p["cls_w"]),
        "cls_b": p["cls_b"].reshape(1, HID),
        "spe_w1t": p["self_pe"]["w1"].T.reshape(2, HID),
        "spe_b1": p["self_pe"]["b1"].reshape(1, HID),
        "spe_g": p["self_pe"]["g"].reshape(1, HID),
        "spe_b": p["self_pe"]["b"].reshape(1, HID),
        "spe_w2t": _bf(p["self_pe"]["w2"].T),
        "spe_b2": p["self_pe"]["b2"].reshape(1, HID),
        "cpe_w1t": p["cross_pe"]["w1"].T.reshape(2, HID),
        "cpe_b1": p["cross_pe"]["b1"].reshape(1, HID),
        "cpe_g": p["cross_pe"]["g"].reshape(1, HID),
        "cpe_b": p["cross_pe"]["b"].reshape(1, HID),
        "cpe_w2t": _bf(p["cross_pe"]["w2"].T),
        "cpe_b2": p["cross_pe"]["b2"].reshape(1, HID),
        "n1_g": p["n1_g"].reshape(1, HID), "n1_b": p["n1_b"].reshape(1, HID),
        "n2_g": p["n2_g"].reshape(1, HID), "n2_b": p["n2_b"].reshape(1, HID),
        "n3_g": p["n3_g"].reshape(1, HID), "n3_b": p["n3_b"].reshape(1, HID),
        "ffn_w1t": _bf(p["ffn_w1"].T), "ffn_b1": p["ffn_b1"].reshape(1, FFN),
        "ffn_w2t": _bf(p["ffn_w2"].T), "ffn_b2": p["ffn_b2"].reshape(1, HID),
    }
    for nm in ("sa", "ca"):
        a = p[nm]
        prep[nm + "_wqt"] = _bf(a["wq"].T)
        prep[nm + "_bq"] = a["bq"].reshape(1, HID)
        prep[nm + "_wkt"] = _bf(a["wk"].T)
        prep[nm + "_bk"] = a["bk"].reshape(1, HID)
        prep[nm + "_wvt"] = _bf(a["wv"].T)
        prep[nm + "_bv"] = a["bv"].reshape(1, HID)
        prep[nm + "_wot"] = _bf(a["wo"].T)
        prep[nm + "_bo"] = a["bo"].reshape(1, HID)
    for name, oc in HEAD_DEFS:
        hp = p["heads"][name]
        prep[name + "_w1t"] = _bf(hp["w1"].T)
        prep[name + "_g"] = hp["g"].reshape(1, 64)
        prep[name + "_b"] = hp["b"].reshape(1, 64)
        prep[name + "_w2t"] = _bf(hp["w2"].T)
        prep[name + "_b2"] = hp["b2"].reshape(1, oc)

    preds = _decoder(lidar, top_idx, top_cls.reshape(P, 1), qpos, bev, prep)

    qhs = jnp.take(masked, top_idx, axis=1)
    return (preds.T[None], qhs[None], dh.reshape(1, NC, H, W))
